# Initial kernel scaffold; baseline (speedup 1.0000x reference)
import jax
import jax.numpy as jnp
from jax.experimental import pallas as pl


def kernel(x, edge_index, batch, params):
    raise NotImplementedError("write your pallas kernel here")


# speedup vs baseline: 9.3560x; 9.3560x over previous
import functools
import math

import jax
import jax.numpy as jnp
from jax import lax
from jax.experimental import pallas as pl
from jax.experimental.pallas import tpu as pltpu
from jax.experimental.pallas import tpu_sc as plsc

_N_NODES = 10000
_N_EDGES = 320000
_DIM = 32
_NG = 256
_NCLS = 2
_INV_BN = 1.0 / math.sqrt(1.0 + 1e-5)

_NC, _NS, _NW = 2, 16, 32
_K = 128
_CH = -(-_N_EDGES // (_NW * _K))
_EPT = _CH * _K
_ZPT = -(-(_N_NODES + 1) // (_NS * 8)) * 8
_ZROWS = _ZPT * _NS


def _make_agg():
    mesh = plsc.VectorSubcoreMesh(core_axis_name="c", subcore_axis_name="s")

    @functools.partial(
        pl.kernel,
        mesh=mesh,
        compiler_params=pltpu.CompilerParams(use_tc_tiling_on_sc=False),
        out_type=jax.ShapeDtypeStruct((_NC, _ZROWS, _DIM), jnp.float32),
        scratch_types=[
            pltpu.VMEM((_CH, _K), jnp.int32),
            pltpu.VMEM((_CH, _K), jnp.int32),
            pltpu.VMEM((_K, _DIM), jnp.float32),
            pltpu.VMEM_SHARED((_ZROWS, _DIM), jnp.float32),
            pltpu.SemaphoreType.DMA,
        ],
    )
    def _agg(src_hbm, dst_hbm, u_hbm, zeros_hbm, out_hbm,
             src_v, dst_v, rows_v, acc_sh, sem):
        cid = lax.axis_index("c")
        sid = lax.axis_index("s")
        wid = cid * _NS + sid
        pltpu.sync_copy(zeros_hbm.at[pl.ds(sid * _ZPT, _ZPT)],
                        acc_sh.at[pl.ds(sid * _ZPT, _ZPT)])
        pltpu.sync_copy(src_hbm.at[wid], src_v)
        pltpu.sync_copy(dst_hbm.at[wid], dst_v)
        plsc.subcore_barrier()

        def body(j, carry):
            pltpu.async_copy(u_hbm.at[src_v.at[j]], rows_v, sem).wait()
            pltpu.sync_copy(rows_v, acc_sh.at[dst_v.at[j]], add=True)
            return carry

        lax.fori_loop(0, _CH, body, 0)
        plsc.subcore_barrier()
        pltpu.sync_copy(acc_sh.at[pl.ds(sid * _ZPT, _ZPT)],
                        out_hbm.at[cid, pl.ds(sid * _ZPT, _ZPT)])

    return _agg


@functools.cache
def _agg_call():
    return _make_agg()


def _mm0_body(x_ref, w_ref, o_ref):
    o_ref[...] = jnp.dot(x_ref[...], w_ref[...],
                         preferred_element_type=jnp.float32)


def _stage_body(u_ref, p_ref, b1_ref, w2_ref, b2_ref, g_ref, bb_ref,
                w1n_ref, o_ref):
    p = (p_ref[0] + p_ref[1])[:_N_NODES]
    z = jnp.maximum(u_ref[...] + p + b1_ref[...], 0.0)
    z = jnp.dot(z, w2_ref[...], preferred_element_type=jnp.float32) + b2_ref[...]
    h = jnp.maximum(z, 0.0) * (g_ref[...] * _INV_BN) + bb_ref[...]
    o_ref[...] = jnp.dot(h, w1n_ref[...], preferred_element_type=jnp.float32)


def _final_body(u_ref, p_ref, b1_ref, w2_ref, b2_ref, g_ref, bb_ref,
                batch_ref, f1w_ref, f1b_ref, f2w_ref, f2b_ref, o_ref):
    p = (p_ref[0] + p_ref[1])[:_N_NODES]
    z = jnp.maximum(u_ref[...] + p + b1_ref[...], 0.0)
    z = jnp.dot(z, w2_ref[...], preferred_element_type=jnp.float32) + b2_ref[...]
    h = jnp.maximum(z, 0.0) * (g_ref[...] * _INV_BN) + bb_ref[...]
    seg = lax.broadcasted_iota(jnp.int32, (_NG, _N_NODES), 0)
    m = (seg == batch_ref[...]).astype(jnp.float32)
    pooled = jnp.dot(m, h, preferred_element_type=jnp.float32)
    g1 = jnp.maximum(
        jnp.dot(pooled, f1w_ref[...], preferred_element_type=jnp.float32)
        + f1b_ref[...], 0.0)
    logits = (jnp.dot(g1, f2w_ref[...], preferred_element_type=jnp.float32)
              + f2b_ref[...])
    mx = jnp.max(logits, axis=-1, keepdims=True)
    lse = mx + jnp.log(jnp.sum(jnp.exp(logits - mx), axis=-1, keepdims=True))
    o_ref[...] = logits - lse


def _shape(s):
    return jax.ShapeDtypeStruct(s, jnp.float32)


def kernel(x, edge_index, batch, params):
    src = edge_index[0]
    dst = edge_index[1]
    pad = _EPT * _NW - _N_EDGES
    src_p = jnp.concatenate(
        [src, jnp.zeros((pad,), jnp.int32)]).reshape(_NW, _CH, _K)
    dst_p = jnp.concatenate(
        [dst, jnp.full((pad,), _N_NODES, jnp.int32)]).reshape(_NW, _CH, _K)
    zeros = jnp.zeros((_ZROWS, _DIM), jnp.float32)
    b2d = lambda v: v.reshape(1, -1)
    batch2d = batch.reshape(1, _N_NODES)

    u = pl.pallas_call(_mm0_body, out_shape=_shape((_N_NODES, _DIM)))(
        x, params['W1_0'])
    for i in range(4):
        p = _agg_call()(src_p, dst_p, u, zeros)
        u = pl.pallas_call(_stage_body, out_shape=_shape((_N_NODES, _DIM)))(
            u, p, b2d(params['b1_%d' % i]), params['W2_%d' % i],
            b2d(params['b2_%d' % i]), b2d(params['bn_g_%d' % i]),
            b2d(params['bn_b_%d' % i]), params['W1_%d' % (i + 1)])
    p = _agg_call()(src_p, dst_p, u, zeros)
    out = pl.pallas_call(_final_body, out_shape=_shape((_NG, _NCLS)))(
        u, p, b2d(params['b1_4']), params['W2_4'], b2d(params['b2_4']),
        b2d(params['bn_g_4']), b2d(params['bn_b_4']), batch2d,
        params['fc1_W'], b2d(params['fc1_b']), params['fc2_W'],
        b2d(params['fc2_b']))
    return out
